# baseline (speedup 1.0000x reference)
import functools

import jax
import jax.numpy as jnp
from jax import lax
from jax.experimental import pallas as pl
from jax.experimental.pallas import tpu as pltpu
from jax.experimental.pallas import tpu_sc as plsc

B = 1024
V = 100000
K = 64
TEMP = 10.0

RB = 8
NW = 32
ROWS_PER_W = B // NW
ELEMS_PER_W = ROWS_PER_W * K
CH = 128
NCH = ELEMS_PER_W // CH
LANES = 16


def _tc_body(lam_ref, nmt_ref, dist_ref, tgt_ref, out_ref, w_ref, oml_ref):
    lam = lam_ref[...]
    oml = 1.0 - lam
    out_ref[...] = oml * nmt_ref[...]

    d = dist_ref[...] * (-1.0 / TEMP)
    m = jnp.max(d, axis=-1, keepdims=True)
    e = jnp.exp(d - m)
    p = (e / jnp.sum(e, axis=-1, keepdims=True)) * lam

    t = tgt_ref[...]
    w = jnp.zeros((RB, K), jnp.float32)
    for j in range(K):
        w = w + jnp.where(t == t[:, j : j + 1], p[:, j : j + 1], 0.0)
    w_ref[...] = w
    oml_ref[...] = jnp.broadcast_to(oml, (RB, K))


_tc_scale = pl.pallas_call(
    _tc_body,
    grid=(B // RB,),
    in_specs=[
        pl.BlockSpec((RB, 1), lambda i: (i, 0)),
        pl.BlockSpec((RB, V), lambda i: (i, 0)),
        pl.BlockSpec((RB, K), lambda i: (i, 0)),
        pl.BlockSpec((RB, K), lambda i: (i, 0)),
    ],
    out_specs=[
        pl.BlockSpec((RB, V), lambda i: (i, 0)),
        pl.BlockSpec((RB, K), lambda i: (i, 0)),
        pl.BlockSpec((RB, K), lambda i: (i, 0)),
    ],
    out_shape=[
        jax.ShapeDtypeStruct((B, V), jnp.float32),
        jax.ShapeDtypeStruct((B, K), jnp.float32),
        jax.ShapeDtypeStruct((B, K), jnp.float32),
    ],
    compiler_params=pltpu.CompilerParams(
        dimension_semantics=("arbitrary",),
    ),
)


@functools.partial(
    pl.kernel,
    mesh=plsc.VectorSubcoreMesh(
        core_axis_name="c", subcore_axis_name="s", num_cores=2, num_subcores=16
    ),
    out_type=(),
    scratch_types=[
        pltpu.VMEM((NCH, CH), jnp.int32),
        pltpu.VMEM((NCH, CH), jnp.float32),
        pltpu.VMEM((NCH, CH), jnp.float32),
        pltpu.VMEM((NCH, CH), jnp.float32),
        pltpu.VMEM((NCH, CH), jnp.float32),
        [pltpu.VMEM((CH,), jnp.int32) for _ in range(NCH)],
        pltpu.SemaphoreType.DMA,
    ],
)
def _sc_scatter(out_ref, nmt_ref, tgt_ref, w_ref, oml_ref,
                tgt_v, w_v, oml_v, g_v, vals_v, idx_refs, sem):
    wid = lax.axis_index("s") * 2 + lax.axis_index("c")
    pltpu.sync_copy(tgt_ref.at[wid], tgt_v)
    pltpu.sync_copy(w_ref.at[wid], w_v)
    pltpu.sync_copy(oml_ref.at[wid], oml_v)

    base_row = wid * ROWS_PER_W
    chunks_per_row = CH // LANES

    for j in range(NCH):
        for q in range(chunks_per_row):
            c = j * chunks_per_row + q
            r_off = c // (K // LANES)
            t16 = tgt_v[j, pl.ds(q * LANES, LANES)]
            idx_refs[j][pl.ds(q * LANES, LANES)] = t16 + (
                base_row + r_off) * V

    gathers = [
        pltpu.async_copy(nmt_ref.at[idx_refs[j]], g_v.at[j], sem)
        for j in range(NCH)
    ]
    for c in gathers:
        c.wait()

    for j in range(NCH):
        for q in range(chunks_per_row):
            sl = (j, pl.ds(q * LANES, LANES))
            vals_v[sl] = oml_v[sl] * g_v[sl] + w_v[sl]

    scatters = [
        pltpu.async_copy(vals_v.at[j], out_ref.at[idx_refs[j]], sem)
        for j in range(NCH)
    ]
    for c in scatters:
        c.wait()


def kernel(nmt_prob, knn_tgt, knn_dist, part_knn_lambda):
    lam2 = part_knn_lambda.reshape(B, 1)
    tgt = knn_tgt.astype(jnp.int32)
    out, w, oml = _tc_scale(lam2, nmt_prob, knn_dist, tgt)
    out_ref = jax.new_ref(out.reshape(B * V))
    _sc_scatter(
        out_ref,
        nmt_prob.reshape(B * V),
        tgt.reshape(NW, NCH, CH),
        w.reshape(NW, NCH, CH),
        oml.reshape(NW, NCH, CH),
    )
    return out_ref[...].reshape(B, V)


# speedup vs baseline: 2.2307x; 2.2307x over previous
import functools

import jax
import jax.numpy as jnp
from jax import lax
from jax.experimental import pallas as pl
from jax.experimental.pallas import tpu as pltpu

B = 1024
V = 100000
K = 64
TEMP = 10.0

RB = 8
VPAD = 100096


def _p_body(lam_ref, dist_ref, p_ref):
    lam = lam_ref[...]
    d = dist_ref[...] * (-1.0 / TEMP)
    m = jnp.max(d, axis=-1, keepdims=True)
    e = jnp.exp(d - m)
    p_ref[...] = (e / jnp.sum(e, axis=-1, keepdims=True)) * lam


_p_kernel = pl.pallas_call(
    _p_body,
    out_shape=jax.ShapeDtypeStruct((B, K), jnp.float32),
)


def _fused_body(lam_ref, tgt_ref, p_ref, nmt_ref, out_ref):
    lam = lam_ref[...]
    out_ref[...] = (1.0 - lam) * nmt_ref[...]

    lanes = lax.broadcasted_iota(jnp.int32, (RB, 128), 1)
    subs = lax.broadcasted_iota(jnp.int32, (RB, 128), 0)

    def apply_k(k, carry):
        for r in range(RB):
            t = tgt_ref[r, k]
            pv = p_ref[r, k]
            c0 = pl.multiple_of((t // 128) * 128, 128)
            l = lax.rem(t, 128)
            blk = out_ref[:, pl.ds(c0, 128)]
            upd = jnp.where((subs == r) & (lanes == l), pv, 0.0)
            out_ref[:, pl.ds(c0, 128)] = blk + upd
        return carry

    lax.fori_loop(0, K, apply_k, 0)


_fused = pl.pallas_call(
    _fused_body,
    grid=(B // RB,),
    in_specs=[
        pl.BlockSpec((RB, 1), lambda i: (i, 0)),
        pl.BlockSpec((RB, K), lambda i: (i, 0), memory_space=pltpu.SMEM),
        pl.BlockSpec((RB, K), lambda i: (i, 0), memory_space=pltpu.SMEM),
        pl.BlockSpec((RB, VPAD), lambda i: (i, 0)),
    ],
    out_specs=pl.BlockSpec((RB, VPAD), lambda i: (i, 0)),
    out_shape=jax.ShapeDtypeStruct((B, V), jnp.float32),
    compiler_params=pltpu.CompilerParams(
        dimension_semantics=("arbitrary",),
    ),
)


def kernel(nmt_prob, knn_tgt, knn_dist, part_knn_lambda):
    lam2 = part_knn_lambda.reshape(B, 1)
    tgt = knn_tgt.astype(jnp.int32)
    p = _p_kernel(lam2, knn_dist)
    return _fused(lam2, tgt, p, nmt_prob)


# speedup vs baseline: 2.3378x; 1.0480x over previous
import jax
import jax.numpy as jnp
from jax import lax
from jax.experimental import pallas as pl
from jax.experimental.pallas import tpu as pltpu

B = 1024
V = 100000
K = 64
TEMP = 10.0

RB = 8
VPAD = 100096


def _fused_body(tgt_s, lam_ref, dist_ref, tgt_ref, nmt_ref, out_ref):
    lam = lam_ref[...]
    out_ref[...] = (1.0 - lam) * nmt_ref[...]

    d = dist_ref[...] * (-1.0 / TEMP)
    m = jnp.max(d, axis=-1, keepdims=True)
    e = jnp.exp(d - m)
    p = (e / jnp.sum(e, axis=-1, keepdims=True)) * lam

    lanes = lax.broadcasted_iota(jnp.int32, (1, 128), 1)

    for r in range(RB):
        trow = tgt_ref[r : r + 1, :]
        tcol = jnp.transpose(trow)
        oh = (lax.rem(tcol, 128) == lanes).astype(jnp.float32)
        mask = (tcol // 128 == trow // 128).astype(jnp.float32)
        mw = mask * p[r : r + 1, :]
        upd = jax.lax.dot_general(
            mw, oh, (((1,), (0,)), ((), ())),
            preferred_element_type=jnp.float32,
        )
        oml = 1.0 - lam[r : r + 1, :]
        for k in range(K):
            c0 = pl.multiple_of((tgt_s[r, k] // 128) * 128, 128)
            out_ref[r : r + 1, pl.ds(c0, 128)] = (
                oml * nmt_ref[r : r + 1, pl.ds(c0, 128)] + upd[k : k + 1, :]
            )


_fused = pl.pallas_call(
    _fused_body,
    grid=(B // RB,),
    in_specs=[
        pl.BlockSpec((RB, K), lambda i: (i, 0), memory_space=pltpu.SMEM),
        pl.BlockSpec((RB, 1), lambda i: (i, 0)),
        pl.BlockSpec((RB, K), lambda i: (i, 0)),
        pl.BlockSpec((RB, K), lambda i: (i, 0)),
        pl.BlockSpec((RB, VPAD), lambda i: (i, 0)),
    ],
    out_specs=pl.BlockSpec((RB, VPAD), lambda i: (i, 0)),
    out_shape=jax.ShapeDtypeStruct((B, V), jnp.float32),
    compiler_params=pltpu.CompilerParams(
        dimension_semantics=("arbitrary",),
    ),
)


def kernel(nmt_prob, knn_tgt, knn_dist, part_knn_lambda):
    lam2 = part_knn_lambda.reshape(B, 1)
    tgt = knn_tgt.astype(jnp.int32)
    return _fused(tgt, lam2, knn_dist, tgt, nmt_prob)


# speedup vs baseline: 2.6737x; 1.1437x over previous
import jax
import jax.numpy as jnp
from jax import lax
from jax.experimental import pallas as pl
from jax.experimental.pallas import tpu as pltpu

B = 1024
V = 100000
K = 64
TEMP = 10.0

RB = 8
VPAD = 100096


def _fused_body(c0_s, lam_ref, dist_ref, tgt_ref, nmt_ref, out_ref):
    lam = lam_ref[...]
    out_ref[...] = (1.0 - lam) * nmt_ref[...]

    d = dist_ref[...] * (-1.0 / TEMP)
    m = jnp.max(d, axis=-1, keepdims=True)
    e = jnp.exp(d - m)
    p = (e / jnp.sum(e, axis=-1, keepdims=True)) * lam

    lanes = lax.broadcasted_iota(jnp.int32, (1, 128), 1)

    upds = []
    for r in range(RB):
        trow = tgt_ref[r : r + 1, :]
        tcol = jnp.transpose(trow)
        oh = (lax.rem(tcol, 128) == lanes).astype(jnp.float32)
        mask = (tcol // 128 == trow // 128).astype(jnp.float32)
        mw = mask * p[r : r + 1, :]
        upds.append(jax.lax.dot_general(
            mw, oh, (((1,), (0,)), ((), ())),
            preferred_element_type=jnp.float32,
        ))

    oml = 1.0 - lam
    for k in range(K):
        for r in range(RB):
            c0 = pl.multiple_of(c0_s[r, k], 128)
            out_ref[r : r + 1, pl.ds(c0, 128)] = (
                oml[r : r + 1, :] * nmt_ref[r : r + 1, pl.ds(c0, 128)]
                + upds[r][k : k + 1, :]
            )


_fused = pl.pallas_call(
    _fused_body,
    grid=(B // RB,),
    in_specs=[
        pl.BlockSpec((RB, K), lambda i: (i, 0), memory_space=pltpu.SMEM),
        pl.BlockSpec((RB, 1), lambda i: (i, 0)),
        pl.BlockSpec((RB, K), lambda i: (i, 0)),
        pl.BlockSpec((RB, K), lambda i: (i, 0)),
        pl.BlockSpec((RB, VPAD), lambda i: (i, 0)),
    ],
    out_specs=pl.BlockSpec((RB, VPAD), lambda i: (i, 0)),
    out_shape=jax.ShapeDtypeStruct((B, V), jnp.float32),
    compiler_params=pltpu.CompilerParams(
        dimension_semantics=("arbitrary",),
    ),
)


def kernel(nmt_prob, knn_tgt, knn_dist, part_knn_lambda):
    lam2 = part_knn_lambda.reshape(B, 1)
    tgt = knn_tgt.astype(jnp.int32)
    c0 = (tgt // 128) * 128
    return _fused(c0, lam2, knn_dist, tgt, nmt_prob)
